# baseline (speedup 1.0000x reference)
import functools

import jax
import jax.numpy as jnp
from jax import lax
from jax.experimental import pallas as pl
from jax.experimental.pallas import tpu as pltpu
from jax.experimental.pallas import tpu_sc as plsc

N = 10000
E = 160000
B = 128
NODE_FDIM = 256
EDGE_FDIM = 16
H = 512
NUM_STEPS = 4

C = 4
CW = H // C
CWP = CW // 2
NC = 2
NS = 16
EPT = E // NS
K = 80
K2 = K // 2
NIT = EPT // K
NPAD = 10240
RPT = NPAD // NS
ZR = 128
RB = 2000
RBU = 2048
EB = 4000


def _bdot(x, w):
    return jnp.dot(x, w, preferred_element_type=jnp.float32)


def _mm_bias_relu_body(x_ref, w_ref, b_ref, o_ref):
    o_ref[...] = jnp.maximum(_bdot(x_ref[...], w_ref[...]) + b_ref[...], 0.0)


def _h0(x, w, b):
    return pl.pallas_call(
        _mm_bias_relu_body,
        grid=(N // RB,),
        in_specs=[
            pl.BlockSpec((RB, NODE_FDIM), lambda i: (i, 0)),
            pl.BlockSpec((NODE_FDIM, H), lambda i: (0, 0)),
            pl.BlockSpec((1, H), lambda i: (0, 0)),
        ],
        out_specs=pl.BlockSpec((RB, H), lambda i: (i, 0)),
        out_shape=jax.ShapeDtypeStruct((N, H), jnp.float32),
    )(x, w, b)


def _pack_bf16_pair(x):
    w = x.shape[1] // 2
    lo = jax.lax.bitcast_convert_type(
        x[:, :w].astype(jnp.bfloat16), jnp.uint16).astype(jnp.uint32)
    hi = jax.lax.bitcast_convert_type(
        x[:, w:].astype(jnp.bfloat16), jnp.uint16).astype(jnp.uint32)
    return jax.lax.bitcast_convert_type(lo | (hi << 16), jnp.int32)


def _mm_chunk_body(x_ref, w_ref, o_ref):
    o_ref[...] = _bdot(x_ref[...], w_ref[...])


def _p_pair(h, w, pair):
    return pl.pallas_call(
        _mm_chunk_body,
        grid=(N // RB, 2),
        in_specs=[
            pl.BlockSpec((RB, H), lambda i, c: (i, 0)),
            pl.BlockSpec((H, CW), lambda i, c: (0, 2 * pair + c)),
        ],
        out_specs=pl.BlockSpec((RB, CW), lambda i, c: (c * (N // RB) + i, 0)),
        out_shape=jax.ShapeDtypeStruct((2 * N, CW), jnp.float32),
    )(h, w)


def _mm_bias_body(x_ref, w_ref, b_ref, o_ref):
    o_ref[...] = jnp.dot(x_ref[...], w_ref[...],
                         preferred_element_type=jnp.float32) + b_ref[...]


def _eterm_chunked(ef, w, b):
    return pl.pallas_call(
        _mm_bias_body,
        grid=(E // EB, C),
        in_specs=[
            pl.BlockSpec((EB, EDGE_FDIM), lambda i, c: (i, 0)),
            pl.BlockSpec((EDGE_FDIM, CW), lambda i, c: (0, c)),
            pl.BlockSpec((1, CW), lambda i, c: (0, c)),
        ],
        out_specs=pl.BlockSpec((EB, CW), lambda i, c: (c * (E // EB) + i, 0)),
        out_shape=jax.ShapeDtypeStruct((C * E, CW), jnp.float32),
    )(ef, w, b)


def _upd_body(h_ref, a_ref, w1_ref, w2_ref, b_ref, o_ref):
    c = pl.program_id(1)

    @pl.when(c == 0)
    def _():
        o_ref[...] = _bdot(h_ref[...], w1_ref[...]) + b_ref[...]

    o_ref[...] += _bdot(a_ref[...], w2_ref[...])

    @pl.when(c == C - 1)
    def _():
        o_ref[...] = jnp.maximum(o_ref[...], 0.0)


def _update(h, agg, w1, w2, b):
    return pl.pallas_call(
        _upd_body,
        grid=(NPAD // RBU, C),
        in_specs=[
            pl.BlockSpec((RBU, H), lambda i, c: (i, 0)),
            pl.BlockSpec((RBU, CW), lambda i, c: (c * (NPAD // RBU) + i, 0)),
            pl.BlockSpec((H, H), lambda i, c: (0, 0)),
            pl.BlockSpec((CW, H), lambda i, c: (c, 0)),
            pl.BlockSpec((1, H), lambda i, c: (0, 0)),
        ],
        out_specs=pl.BlockSpec((RBU, H), lambda i, c: (i, 0)),
        out_shape=jax.ShapeDtypeStruct((N, H), jnp.float32),
    )(h, agg, w1, w2, b)


def _pool_body(bi_ref, h_ref, o_ref, acc_ref, cnt_ref):
    i = pl.program_id(0)
    bi = bi_ref[0, 0, :]
    onehot = (bi[:, None] == lax.broadcasted_iota(jnp.int32, (1, B), 1)
              ).astype(jnp.float32)
    psum = lax.dot_general(onehot, h_ref[...],
                           (((0,), (0,)), ((), ())),
                           preferred_element_type=jnp.float32)
    pcnt = jnp.sum(onehot, axis=0)[None, :]

    @pl.when(i == 0)
    def _():
        acc_ref[...] = jnp.zeros_like(acc_ref)
        cnt_ref[...] = jnp.zeros_like(cnt_ref)

    acc_ref[...] += psum
    cnt_ref[...] += pcnt

    @pl.when(i == (N // RB) - 1)
    def _():
        o_ref[...] = acc_ref[...] / jnp.maximum(cnt_ref[...], 1.0).T


def _pool(h, batch_idx3):
    return pl.pallas_call(
        _pool_body,
        grid=(N // RB,),
        in_specs=[
            pl.BlockSpec((1, 1, RB), lambda i: (i, 0, 0)),
            pl.BlockSpec((RB, H), lambda i: (i, 0)),
        ],
        out_specs=pl.BlockSpec((B, H), lambda i: (0, 0)),
        out_shape=jax.ShapeDtypeStruct((B, H), jnp.float32),
        scratch_shapes=[
            pltpu.VMEM((B, H), jnp.float32),
            pltpu.VMEM((1, B), jnp.float32),
        ],
    )(batch_idx3, h)


def _edge_sc_body(p_a, p_b, et_hbm, src_hbm, dst_hbm, agg_hbm,
                  idx_v, dst_v, rows_v, et_v, agg_sh, sem_i, sem_g, sem_s):
    cid = lax.axis_index("c")
    sid = lax.axis_index("s")
    ebase = sid * EPT

    def zbody(e, carry):
        for j in range(CW // 16):
            rows_v[0][e, pl.ds(j * 16, 16)] = jnp.zeros((16,), jnp.float32)
        return carry
    lax.fori_loop(0, K, zbody, 0)

    def fire_load(x, t, chunk):
        base = ebase + jnp.minimum(t, NIT - 1) * K
        pltpu.async_copy(src_hbm.at[pl.ds(base, K)], idx_v[x], sem_i[x])
        pltpu.async_copy(dst_hbm.at[pl.ds(base, K)], dst_v[x], sem_i[x])
        pltpu.async_copy(et_hbm.at[pl.ds(chunk * E + base, K)], et_v[x],
                         sem_i[x])

    def wait_load(x):
        pltpu.make_async_copy(src_hbm.at[pl.ds(0, K)], idx_v[x],
                              sem_i[x]).wait()
        pltpu.make_async_copy(dst_hbm.at[pl.ds(0, K)], dst_v[x],
                              sem_i[x]).wait()
        pltpu.make_async_copy(et_hbm.at[pl.ds(0, K)], et_v[x],
                              sem_i[x]).wait()

    def offset(x):
        off = cid * N
        for j in range(K // 16):
            s = pl.ds(j * 16, 16)
            idx_v[x][s] = idx_v[x][s] + off

    def fire_gather(x, p_hbm):
        pltpu.async_copy(p_hbm.at[idx_v[x]], rows_v[x], sem_g[x])

    def wait_gather(x, p_hbm):
        pltpu.make_async_copy(p_hbm.at[pl.ds(0, K)], rows_v[x],
                              sem_g[x]).wait()

    def compute(x):
        def cbody(e, carry):
            for j in range(CW // 16):
                s = pl.ds(j * 16, 16)
                rows_v[x][e, s] = jnp.maximum(
                    rows_v[x][e, s] + et_v[x][e, s], 0.0)
            return carry
        lax.fori_loop(0, K, cbody, 0, unroll=False)

    def fire_scat(x):
        pltpu.async_copy(rows_v[x], agg_sh.at[dst_v[x]], sem_s[x], add=True)

    def wait_scat(x):
        pltpu.make_async_copy(rows_v[x], agg_sh.at[pl.ds(0, K)],
                              sem_s[x]).wait()

    for cc in range(2):
        p_hbm = p_a if cc == 0 else p_b
        chunk = 2 * cc + cid
        for kk in range(RPT // K):
            pltpu.sync_copy(rows_v[0],
                            agg_sh.at[pl.ds(sid * RPT + kk * K, K)])
        plsc.subcore_barrier()

        fire_load(0, 0, chunk)
        wait_load(0)
        offset(0)
        fire_gather(0, p_hbm)
        fire_load(1, 1, chunk)

        def pair_body(u, _):
            t0 = 2 * u
            wait_load(1)
            offset(1)
            fire_gather(1, p_hbm)
            wait_gather(0, p_hbm)
            compute(0)
            fire_scat(0)
            wait_gather(1, p_hbm)
            compute(1)
            wait_scat(0)
            fire_load(0, t0 + 2, chunk)
            fire_scat(1)
            wait_load(0)
            offset(0)
            fire_gather(0, p_hbm)
            wait_scat(1)
            fire_load(1, t0 + 3, chunk)
            return _

        lax.fori_loop(0, (NIT - 1) // 2, pair_body, 0, unroll=False)
        wait_gather(0, p_hbm)
        compute(0)
        fire_scat(0)
        wait_scat(0)
        wait_load(1)
        plsc.subcore_barrier()
        for kk in range(RPT // ZR):
            off = sid * RPT + kk * ZR
            pltpu.sync_copy(agg_sh.at[pl.ds(off, ZR)],
                            agg_hbm.at[pl.ds(chunk * NPAD + off, ZR)])
        plsc.subcore_barrier()
        if cc == 0:
            lax.fori_loop(0, K, zbody, 0)


def _edge_sc(pa, pb, et, src, dst):
    mesh = plsc.VectorSubcoreMesh(core_axis_name="c", subcore_axis_name="s")
    f = functools.partial(
        pl.kernel,
        out_type=jax.ShapeDtypeStruct((C * NPAD, CW), jnp.float32),
        mesh=mesh,
        scratch_types=[
            [pltpu.VMEM((K,), jnp.int32) for _ in range(2)],
            [pltpu.VMEM((K,), jnp.int32) for _ in range(2)],
            [pltpu.VMEM((K, CW), jnp.float32) for _ in range(2)],
            [pltpu.VMEM((K, CW), jnp.float32) for _ in range(2)],
            pltpu.VMEM_SHARED((NPAD, CW), jnp.float32),
            [pltpu.SemaphoreType.DMA for _ in range(2)],
            [pltpu.SemaphoreType.DMA for _ in range(2)],
            [pltpu.SemaphoreType.DMA for _ in range(2)],
        ],
        name="edge_sc",
    )(_edge_sc_body)
    return f(pa, pb, et, src, dst)


def kernel(node_features, edge_features, W_in, b_in, W_msg, b_msg,
           W_upd, b_upd, edge_index, batch_indices):
    src = edge_index[0]
    dst = edge_index[1]
    Wm_h = W_msg[:H]
    Wm_e = W_msg[H:]
    Wu1 = W_upd[:H]
    Wu2 = W_upd[H:]

    h = _h0(node_features, W_in, b_in.reshape(1, H))
    et = _eterm_chunked(edge_features, Wm_e, b_msg.reshape(1, H))
    for _ in range(NUM_STEPS):
        pa = _p_pair(h, Wm_h, 0)
        pb = _p_pair(h, Wm_h, 1)
        agg = _edge_sc(pa, pb, et, src, dst)
        h = _update(h, agg, Wu1, Wu2, b_upd.reshape(1, H))
    return _pool(h, batch_indices.reshape(N // RB, 1, RB))


# speedup vs baseline: 1.0338x; 1.0338x over previous
import functools

import jax
import jax.numpy as jnp
from jax import lax
from jax.experimental import pallas as pl
from jax.experimental.pallas import tpu as pltpu
from jax.experimental.pallas import tpu_sc as plsc

N = 10000
E = 160000
B = 128
NODE_FDIM = 256
EDGE_FDIM = 16
H = 512
NUM_STEPS = 4

C = 4
CW = H // C
CWP = CW // 2
NC = 2
NS = 16
EPT = E // NS
K = 80
K2 = K // 2
NIT = EPT // K
NPAD = 10240
RPT = NPAD // NS
ZR = 128
RB = 2000
RBU = 2048
EB = 4000


def _bdot(x, w):
    return jnp.dot(x, w, preferred_element_type=jnp.float32)


def _mm_bias_relu_body(x_ref, w_ref, b_ref, o_ref):
    o_ref[...] = jnp.maximum(_bdot(x_ref[...], w_ref[...]) + b_ref[...], 0.0)


def _h0(x, w, b):
    return pl.pallas_call(
        _mm_bias_relu_body,
        grid=(N // RB,),
        in_specs=[
            pl.BlockSpec((RB, NODE_FDIM), lambda i: (i, 0)),
            pl.BlockSpec((NODE_FDIM, H), lambda i: (0, 0)),
            pl.BlockSpec((1, H), lambda i: (0, 0)),
        ],
        out_specs=pl.BlockSpec((RB, H), lambda i: (i, 0)),
        out_shape=jax.ShapeDtypeStruct((N, H), jnp.float32),
    )(x, w, b)


def _pack_bf16_pair(x):
    w = x.shape[1] // 2
    lo = jax.lax.bitcast_convert_type(
        x[:, :w].astype(jnp.bfloat16), jnp.uint16).astype(jnp.uint32)
    hi = jax.lax.bitcast_convert_type(
        x[:, w:].astype(jnp.bfloat16), jnp.uint16).astype(jnp.uint32)
    return jax.lax.bitcast_convert_type(lo | (hi << 16), jnp.int32)


def _mm_chunk_body(x_ref, w_ref, o_ref):
    o_ref[...] = _bdot(x_ref[...], w_ref[...])


def _p_pair(h, w, pair):
    return pl.pallas_call(
        _mm_chunk_body,
        grid=(N // RB, 2),
        in_specs=[
            pl.BlockSpec((RB, H), lambda i, c: (i, 0)),
            pl.BlockSpec((H, CW), lambda i, c: (0, 2 * pair + c)),
        ],
        out_specs=pl.BlockSpec((RB, CW), lambda i, c: (c * (N // RB) + i, 0)),
        out_shape=jax.ShapeDtypeStruct((2 * N, CW), jnp.float32),
    )(h, w)


def _mm_bias_body(x_ref, w_ref, b_ref, o_ref):
    o_ref[...] = jnp.dot(x_ref[...], w_ref[...],
                         preferred_element_type=jnp.float32) + b_ref[...]


def _eterm_chunked(ef, w, b):
    return pl.pallas_call(
        _mm_bias_body,
        grid=(E // EB, C),
        in_specs=[
            pl.BlockSpec((EB, EDGE_FDIM), lambda i, c: (i, 0)),
            pl.BlockSpec((EDGE_FDIM, CW), lambda i, c: (0, c)),
            pl.BlockSpec((1, CW), lambda i, c: (0, c)),
        ],
        out_specs=pl.BlockSpec((EB, CW), lambda i, c: (c * (E // EB) + i, 0)),
        out_shape=jax.ShapeDtypeStruct((C * E, CW), jnp.float32),
    )(ef, w, b)


def _updp_body(h_ref, a_ref, w1_ref, w2_ref, b_ref, o_ref):
    c = pl.program_id(1)

    @pl.when(c == 0)
    def _():
        o_ref[...] = _bdot(h_ref[...], w1_ref[...]) + b_ref[...]

    o_ref[...] += _bdot(a_ref[...], w2_ref[...])


def _update_partial(h, agg2, w1, w2a, b):
    return pl.pallas_call(
        _updp_body,
        grid=(NPAD // RBU, 2),
        in_specs=[
            pl.BlockSpec((RBU, H), lambda i, c: (i, 0)),
            pl.BlockSpec((RBU, CW), lambda i, c: (c * (NPAD // RBU) + i, 0)),
            pl.BlockSpec((H, H), lambda i, c: (0, 0)),
            pl.BlockSpec((CW, H), lambda i, c: (c, 0)),
            pl.BlockSpec((1, H), lambda i, c: (0, 0)),
        ],
        out_specs=pl.BlockSpec((RBU, H), lambda i, c: (i, 0)),
        out_shape=jax.ShapeDtypeStruct((N, H), jnp.float32),
    )(h, agg2, w1, w2a, b)


def _updf_body(p_ref, a_ref, w2_ref, o_ref):
    c = pl.program_id(1)

    @pl.when(c == 0)
    def _():
        o_ref[...] = p_ref[...]

    o_ref[...] += _bdot(a_ref[...], w2_ref[...])

    @pl.when(c == 1)
    def _():
        o_ref[...] = jnp.maximum(o_ref[...], 0.0)


def _update_final(pre, agg2, w2b):
    return pl.pallas_call(
        _updf_body,
        grid=(NPAD // RBU, 2),
        in_specs=[
            pl.BlockSpec((RBU, H), lambda i, c: (i, 0)),
            pl.BlockSpec((RBU, CW), lambda i, c: (c * (NPAD // RBU) + i, 0)),
            pl.BlockSpec((CW, H), lambda i, c: (c, 0)),
        ],
        out_specs=pl.BlockSpec((RBU, H), lambda i, c: (i, 0)),
        out_shape=jax.ShapeDtypeStruct((N, H), jnp.float32),
    )(pre, agg2, w2b)


def _pool_body(bi_ref, h_ref, o_ref, acc_ref, cnt_ref):
    i = pl.program_id(0)
    bi = bi_ref[0, 0, :]
    onehot = (bi[:, None] == lax.broadcasted_iota(jnp.int32, (1, B), 1)
              ).astype(jnp.float32)
    psum = lax.dot_general(onehot, h_ref[...],
                           (((0,), (0,)), ((), ())),
                           preferred_element_type=jnp.float32)
    pcnt = jnp.sum(onehot, axis=0)[None, :]

    @pl.when(i == 0)
    def _():
        acc_ref[...] = jnp.zeros_like(acc_ref)
        cnt_ref[...] = jnp.zeros_like(cnt_ref)

    acc_ref[...] += psum
    cnt_ref[...] += pcnt

    @pl.when(i == (N // RB) - 1)
    def _():
        o_ref[...] = acc_ref[...] / jnp.maximum(cnt_ref[...], 1.0).T


def _pool(h, batch_idx3):
    return pl.pallas_call(
        _pool_body,
        grid=(N // RB,),
        in_specs=[
            pl.BlockSpec((1, 1, RB), lambda i: (i, 0, 0)),
            pl.BlockSpec((RB, H), lambda i: (i, 0)),
        ],
        out_specs=pl.BlockSpec((B, H), lambda i: (0, 0)),
        out_shape=jax.ShapeDtypeStruct((B, H), jnp.float32),
        scratch_shapes=[
            pltpu.VMEM((B, H), jnp.float32),
            pltpu.VMEM((1, B), jnp.float32),
        ],
    )(batch_idx3, h)


def _edge_sc_body(pair, p_hbm, et_hbm, src_hbm, dst_hbm, agg_hbm,
                  idx_v, dst_v, rows_v, et_v, agg_sh, sem_i, sem_g, sem_s):
    cid = lax.axis_index("c")
    sid = lax.axis_index("s")
    ebase = sid * EPT

    def zbody(e, carry):
        for j in range(CW // 16):
            rows_v[0][e, pl.ds(j * 16, 16)] = jnp.zeros((16,), jnp.float32)
        return carry
    lax.fori_loop(0, K, zbody, 0)

    def fire_load(x, t, chunk):
        base = ebase + jnp.minimum(t, NIT - 1) * K
        pltpu.async_copy(src_hbm.at[pl.ds(base, K)], idx_v[x], sem_i[x])
        pltpu.async_copy(dst_hbm.at[pl.ds(base, K)], dst_v[x], sem_i[x])
        pltpu.async_copy(et_hbm.at[pl.ds(chunk * E + base, K)], et_v[x],
                         sem_i[x])

    def wait_load(x):
        pltpu.make_async_copy(src_hbm.at[pl.ds(0, K)], idx_v[x],
                              sem_i[x]).wait()
        pltpu.make_async_copy(dst_hbm.at[pl.ds(0, K)], dst_v[x],
                              sem_i[x]).wait()
        pltpu.make_async_copy(et_hbm.at[pl.ds(0, K)], et_v[x],
                              sem_i[x]).wait()

    def offset(x):
        off = cid * N
        for j in range(K // 16):
            s = pl.ds(j * 16, 16)
            idx_v[x][s] = idx_v[x][s] + off

    def fire_gather(x):
        pltpu.async_copy(p_hbm.at[idx_v[x]], rows_v[x], sem_g[x])

    def wait_gather(x):
        pltpu.make_async_copy(p_hbm.at[pl.ds(0, K)], rows_v[x],
                              sem_g[x]).wait()

    def compute(x):
        def cbody(e, carry):
            for j in range(CW // 16):
                s = pl.ds(j * 16, 16)
                rows_v[x][e, s] = jnp.maximum(
                    rows_v[x][e, s] + et_v[x][e, s], 0.0)
            return carry
        lax.fori_loop(0, K, cbody, 0, unroll=False)

    def fire_scat(x):
        pltpu.async_copy(rows_v[x], agg_sh.at[dst_v[x]], sem_s[x], add=True)

    def wait_scat(x):
        pltpu.make_async_copy(rows_v[x], agg_sh.at[pl.ds(0, K)],
                              sem_s[x]).wait()

    chunk = 2 * pair + cid
    for kk in range(RPT // K):
        pltpu.sync_copy(rows_v[0],
                        agg_sh.at[pl.ds(sid * RPT + kk * K, K)])
    plsc.subcore_barrier()

    fire_load(0, 0, chunk)
    wait_load(0)
    offset(0)
    fire_gather(0)
    fire_load(1, 1, chunk)

    def pair_body(u, _):
        t0 = 2 * u
        wait_load(1)
        offset(1)
        fire_gather(1)
        wait_gather(0)
        compute(0)
        fire_scat(0)
        wait_gather(1)
        compute(1)
        wait_scat(0)
        fire_load(0, t0 + 2, chunk)
        fire_scat(1)
        wait_load(0)
        offset(0)
        fire_gather(0)
        wait_scat(1)
        fire_load(1, t0 + 3, chunk)
        return _

    lax.fori_loop(0, (NIT - 1) // 2, pair_body, 0, unroll=False)
    wait_gather(0)
    compute(0)
    fire_scat(0)
    wait_scat(0)
    wait_load(1)
    plsc.subcore_barrier()
    for kk in range(RPT // ZR):
        off = sid * RPT + kk * ZR
        pltpu.sync_copy(agg_sh.at[pl.ds(off, ZR)],
                        agg_hbm.at[pl.ds(cid * NPAD + off, ZR)])
    plsc.subcore_barrier()


def _edge_sc(p2, et, src, dst, pair):
    mesh = plsc.VectorSubcoreMesh(core_axis_name="c", subcore_axis_name="s")
    f = functools.partial(
        pl.kernel,
        out_type=jax.ShapeDtypeStruct((2 * NPAD, CW), jnp.float32),
        mesh=mesh,
        scratch_types=[
            [pltpu.VMEM((K,), jnp.int32) for _ in range(2)],
            [pltpu.VMEM((K,), jnp.int32) for _ in range(2)],
            [pltpu.VMEM((K, CW), jnp.float32) for _ in range(2)],
            [pltpu.VMEM((K, CW), jnp.float32) for _ in range(2)],
            pltpu.VMEM_SHARED((NPAD, CW), jnp.float32),
            [pltpu.SemaphoreType.DMA for _ in range(2)],
            [pltpu.SemaphoreType.DMA for _ in range(2)],
            [pltpu.SemaphoreType.DMA for _ in range(2)],
        ],
        name=f"edge_sc_pair{pair}",
    )(functools.partial(_edge_sc_body, pair))
    return f(p2, et, src, dst)


def kernel(node_features, edge_features, W_in, b_in, W_msg, b_msg,
           W_upd, b_upd, edge_index, batch_indices):
    src = edge_index[0]
    dst = edge_index[1]
    Wm_h = W_msg[:H]
    Wm_e = W_msg[H:]
    Wu1 = W_upd[:H]
    Wu2 = W_upd[H:]

    h = _h0(node_features, W_in, b_in.reshape(1, H))
    et = _eterm_chunked(edge_features, Wm_e, b_msg.reshape(1, H))
    for _ in range(NUM_STEPS):
        pa = _p_pair(h, Wm_h, 0)
        agg_a = _edge_sc(pa, et, src, dst, 0)
        pb = _p_pair(h, Wm_h, 1)
        agg_b = _edge_sc(pb, et, src, dst, 1)
        pre = _update_partial(h, agg_a, Wu1, Wu2[:2 * CW],
                              b_upd.reshape(1, H))
        h = _update_final(pre, agg_b, Wu2[2 * CW:])
    return _pool(h, batch_indices.reshape(N // RB, 1, RB))
